# Initial kernel scaffold; baseline (speedup 1.0000x reference)
import jax
import jax.numpy as jnp
from jax.experimental import pallas as pl


def kernel(features, tables):
    raise NotImplementedError("write your pallas kernel here")


# speedup vs baseline: 1.2102x; 1.2102x over previous
import functools

import jax
import jax.numpy as jnp
from jax import lax
from jax.experimental import pallas as pl
from jax.experimental.pallas import tpu as pltpu
from jax.experimental.pallas import tpu_sc as plsc

NC = 2
NS = 16
LANES = 16
NW = NC * NS
CHUNK = 128
NBUF = 8


def _embed_kernel(n_fields, vocab, dim, n_rows):
    per_w = n_rows // NW
    n_chunks = per_w // CHUNK
    n_groups = n_chunks // NBUF
    mesh = plsc.VectorSubcoreMesh(core_axis_name="c", subcore_axis_name="s")

    @functools.partial(
        pl.kernel,
        mesh=mesh,
        compiler_params=pltpu.CompilerParams(use_tc_tiling_on_sc=False),
        out_type=jax.ShapeDtypeStruct((n_rows, dim), jnp.float32),
        scratch_types=(
            [pltpu.VMEM((per_w,), jnp.int32)]
            + [pltpu.VMEM((CHUNK, dim), jnp.float32) for _ in range(NBUF)]
            + [pltpu.SemaphoreType.DMA for _ in range(2 * NBUF)]
        ),
    )
    def k(table_hbm, feats_hbm, out_hbm, idx_v, *bufs_sems):
        rows = bufs_sems[:NBUF]
        gsem = bufs_sems[NBUF:2 * NBUF]
        osem = bufs_sems[2 * NBUF:]

        wid = lax.axis_index("s") * NC + lax.axis_index("c")
        base = wid * per_w

        pltpu.sync_copy(feats_hbm.at[pl.ds(base, per_w)], idx_v)

        lane = jnp.arange(LANES, dtype=jnp.int32)

        def adj_body(t, carry):
            s = t * LANES
            pos = lane + s
            off = (pos % n_fields) * vocab
            idx_v[pl.ds(s, LANES)] = idx_v[pl.ds(s, LANES)] + off
            return carry

        lax.fori_loop(0, per_w // LANES, adj_body, 0)

        def gather(j, b):
            pltpu.make_async_copy(
                table_hbm.at[idx_v.at[pl.ds(j * CHUNK, CHUNK)]],
                rows[b], gsem[b],
            ).start()

        def drain_and_put(j, b):
            pltpu.make_async_copy(
                table_hbm.at[idx_v.at[pl.ds(j * CHUNK, CHUNK)]],
                rows[b], gsem[b],
            ).wait()
            pltpu.make_async_copy(
                rows[b], out_hbm.at[pl.ds(base + j * CHUNK, CHUNK)], osem[b],
            ).start()

        def out_wait(j, b):
            pltpu.make_async_copy(
                rows[b], out_hbm.at[pl.ds(base + j * CHUNK, CHUNK)], osem[b],
            ).wait()

        def group(g, carry):
            for b in range(NBUF):
                j = g * NBUF + b

                @pl.when(g > 0)
                def _():
                    out_wait(j - NBUF, b)

                gather(j, b)
            for b in range(NBUF):
                drain_and_put(g * NBUF + b, b)
            return carry

        lax.fori_loop(0, n_groups, group, 0)
        for b in range(NBUF):
            out_wait((n_groups - 1) * NBUF + b, b)

    return k


def kernel(features, tables):
    b, f = features.shape
    f2, vocab, dim = tables.shape
    assert f == f2
    n_rows = b * f
    assert n_rows % (NW * CHUNK * NBUF) == 0 and (n_rows // NW) % f == 0

    table_flat = tables.reshape(f * vocab, dim)
    feats_flat = features.astype(jnp.int32).reshape(n_rows)
    out = _embed_kernel(f, vocab, dim, n_rows)(table_flat, feats_flat)
    return out.reshape(b, f * dim)


# speedup vs baseline: 1.2184x; 1.0067x over previous
import functools

import jax
import jax.numpy as jnp
from jax import lax
from jax.experimental import pallas as pl
from jax.experimental.pallas import tpu as pltpu
from jax.experimental.pallas import tpu_sc as plsc

NC = 2
NS = 16
NW = NC * NS
CHUNK = 128
NBUF = 8


def _embed_kernel(n_fields, vocab, dim, batch):
    n_rows = n_fields * batch
    chunks_per_field = batch // CHUNK
    per_w = (n_rows // CHUNK) // NW
    n_groups = per_w // NBUF
    mesh = plsc.VectorSubcoreMesh(core_axis_name="c", subcore_axis_name="s")

    @functools.partial(
        pl.kernel,
        mesh=mesh,
        compiler_params=pltpu.CompilerParams(use_tc_tiling_on_sc=False),
        out_type=jax.ShapeDtypeStruct((batch, n_fields * dim), jnp.float32),
        scratch_types=(
            [pltpu.VMEM((per_w * CHUNK,), jnp.int32)]
            + [pltpu.VMEM((CHUNK, dim), jnp.float32) for _ in range(NBUF)]
            + [pltpu.SemaphoreType.DMA for _ in range(2 * NBUF)]
        ),
    )
    def k(tables_hbm, featsT_hbm, out_hbm, idx_v, *bufs_sems):
        rows = bufs_sems[:NBUF]
        gsem = bufs_sems[NBUF:2 * NBUF]
        osem = bufs_sems[2 * NBUF:]

        wid = lax.axis_index("s") * NC + lax.axis_index("c")
        c0 = wid * per_w

        pltpu.sync_copy(featsT_hbm.at[pl.ds(c0 * CHUNK, per_w * CHUNK)], idx_v)

        def out_slice(k_):
            c = c0 + k_
            f = c // chunks_per_field
            b0 = (c % chunks_per_field) * CHUNK
            return out_hbm.at[pl.ds(b0, CHUNK), pl.ds(f * dim, dim)]

        def gather(k_, b):
            c = c0 + k_
            f = c // chunks_per_field
            pltpu.make_async_copy(
                tables_hbm.at[f].at[idx_v.at[pl.ds(k_ * CHUNK, CHUNK)]],
                rows[b], gsem[b],
            ).start()

        def drain_and_put(k_, b):
            pltpu.make_async_copy(
                tables_hbm.at[0].at[idx_v.at[pl.ds(k_ * CHUNK, CHUNK)]],
                rows[b], gsem[b],
            ).wait()
            pltpu.make_async_copy(rows[b], out_slice(k_), osem[b]).start()

        def out_wait(k_, b):
            pltpu.make_async_copy(rows[b], out_slice(k_), osem[b]).wait()

        def group(g, carry):
            for b in range(NBUF):
                k_ = g * NBUF + b

                @pl.when(g > 0)
                def _():
                    out_wait(k_ - NBUF, b)

                gather(k_, b)
            for b in range(NBUF):
                drain_and_put(g * NBUF + b, b)
            return carry

        lax.fori_loop(0, n_groups, group, 0)
        for b in range(NBUF):
            out_wait((n_groups - 1) * NBUF + b, b)

    return k


def kernel(features, tables):
    b, f = features.shape
    f2, vocab, dim = tables.shape
    assert f == f2
    n_chunks = b * f // CHUNK
    assert b % CHUNK == 0 and n_chunks % (NW * NBUF) == 0

    feats_t = features.astype(jnp.int32).T.reshape(b * f)
    return _embed_kernel(f, vocab, dim, b)(tables, feats_t)
